# Initial kernel scaffold; baseline (speedup 1.0000x reference)
import jax
import jax.numpy as jnp
from jax.experimental import pallas as pl


def kernel(x, edge_index, W, b):
    raise NotImplementedError("write your pallas kernel here")


# speedup vs baseline: 28.6123x; 28.6123x over previous
import functools

import jax
import jax.numpy as jnp
from jax import lax
from jax.experimental import pallas as pl
from jax.experimental.pallas import tpu as pltpu
from jax.experimental.pallas import tpu_sc as plsc

NC = 2
NS = 16
NL = 16
NW = NC * NS
CW = 128


def _sc_mesh():
    return plsc.VectorSubcoreMesh(
        core_axis_name="c", subcore_axis_name="s", num_cores=NC, num_subcores=NS
    )


def _make_deg_kernel(N_pad, CH):
    n_per_tile = N_pad // NS

    @functools.partial(
        pl.kernel,
        out_type=jax.ShapeDtypeStruct((NC, N_pad), jnp.float32),
        mesh=_sc_mesh(),
        scratch_types=[
            pltpu.VMEM((CH, CW), jnp.int32),
            pltpu.VMEM((CW,), jnp.float32),
            pltpu.VMEM((n_per_tile,), jnp.float32),
            pltpu.VMEM_SHARED((N_pad,), jnp.float32),
        ],
    )
    def deg_kernel(dst_hbm, out_hbm, idx_v, ones_v, zeros_v, acc_sh):
        c = lax.axis_index("c")
        s = lax.axis_index("s")
        wid = c * NS + s

        def fill_ones(i, _):
            ones_v[pl.ds(i * NL, NL)] = jnp.ones((NL,), jnp.float32)
            return 0

        lax.fori_loop(0, CW // NL, fill_ones, 0)

        def fill_zeros(i, _):
            zeros_v[pl.ds(i * NL, NL)] = jnp.zeros((NL,), jnp.float32)
            return 0

        lax.fori_loop(0, n_per_tile // NL, fill_zeros, 0)
        pltpu.sync_copy(zeros_v, acc_sh.at[pl.ds(s * n_per_tile, n_per_tile)])
        pltpu.sync_copy(dst_hbm.at[wid], idx_v)
        plsc.subcore_barrier()

        def body(j, _):
            pltpu.sync_copy(ones_v, acc_sh.at[idx_v.at[j]], add=True)
            return 0

        lax.fori_loop(0, CH, body, 0)
        plsc.subcore_barrier()
        sl = pl.ds(s * n_per_tile, n_per_tile)
        pltpu.sync_copy(acc_sh.at[sl], out_hbm.at[c, sl])

    return deg_kernel


def _make_agg_kernel(N_pad, CH, D):
    n_per_tile = N_pad // NS

    @functools.partial(
        pl.kernel,
        out_type=jax.ShapeDtypeStruct((NC, N_pad, D), jnp.float32),
        mesh=_sc_mesh(),
        scratch_types=[
            pltpu.VMEM((CH, CW), jnp.int32),
            pltpu.VMEM((CH, CW), jnp.int32),
            pltpu.VMEM((CW, D), jnp.float32),
            pltpu.VMEM_SHARED((N_pad, D), jnp.float32),
            pltpu.SemaphoreType.DMA,
        ],
    )
    def agg_kernel(src_hbm, dst_hbm, h2_hbm, out_hbm, si_v, di_v, rows_v,
                   acc_sh, sem):
        c = lax.axis_index("c")
        s = lax.axis_index("s")
        wid = c * NS + s
        cols = D // NL

        def fill_zeros(i, _):
            rows_v[i // cols, pl.ds((i % cols) * NL, NL)] = jnp.zeros(
                (NL,), jnp.float32)
            return 0

        lax.fori_loop(0, CW * cols, fill_zeros, 0)

        def zero_acc(i, _):
            pltpu.sync_copy(
                rows_v, acc_sh.at[pl.ds(s * n_per_tile + i * CW, CW)])
            return 0

        lax.fori_loop(0, n_per_tile // CW, zero_acc, 0)
        pltpu.sync_copy(src_hbm.at[wid], si_v)
        pltpu.sync_copy(dst_hbm.at[wid], di_v)
        plsc.subcore_barrier()

        def body(j, _):
            pltpu.async_copy(h2_hbm.at[si_v.at[j]], rows_v, sem).wait()
            pltpu.sync_copy(rows_v, acc_sh.at[di_v.at[j]], add=True)
            return 0

        lax.fori_loop(0, CH, body, 0)
        plsc.subcore_barrier()
        sl = pl.ds(s * n_per_tile, n_per_tile)
        pltpu.sync_copy(acc_sh.at[sl], out_hbm.at[c, sl])

    return agg_kernel


def _prep_call(x_pad, W, deg2t, block_rows):
    N_pad, D_in = x_pad.shape
    D_out = W.shape[1]

    def body(x_ref, w_ref, deg_ref, h2_ref):
        deg = deg_ref[:, 0:1] + deg_ref[:, 1:2] + 1.0
        dis = lax.rsqrt(deg)
        h = jnp.dot(x_ref[...], w_ref[...], preferred_element_type=jnp.float32)
        h2_ref[...] = h * dis

    return pl.pallas_call(
        body,
        grid=(N_pad // block_rows,),
        in_specs=[
            pl.BlockSpec((block_rows, D_in), lambda i: (i, 0)),
            pl.BlockSpec((D_in, D_out), lambda i: (0, 0)),
            pl.BlockSpec((block_rows, NC), lambda i: (i, 0)),
        ],
        out_specs=pl.BlockSpec((block_rows, D_out), lambda i: (i, 0)),
        out_shape=jax.ShapeDtypeStruct((N_pad, D_out), jnp.float32),
    )(x_pad, W, deg2t)


def _combine_call(acc, h2, deg2t, b2d, N, block_rows):
    _, N_pad, D = acc.shape

    def body(a0_ref, a1_ref, h2_ref, deg_ref, b_ref, o_ref):
        deg = deg_ref[:, 0:1] + deg_ref[:, 1:2] + 1.0
        dis = lax.rsqrt(deg)
        o_ref[...] = dis * (a0_ref[0] + a1_ref[0] + h2_ref[...]) + b_ref[...]

    return pl.pallas_call(
        body,
        grid=(N_pad // block_rows,),
        in_specs=[
            pl.BlockSpec((1, block_rows, D), lambda i: (0, i, 0)),
            pl.BlockSpec((1, block_rows, D), lambda i: (1, i, 0)),
            pl.BlockSpec((block_rows, D), lambda i: (i, 0)),
            pl.BlockSpec((block_rows, NC), lambda i: (i, 0)),
            pl.BlockSpec((1, D), lambda i: (0, 0)),
        ],
        out_specs=pl.BlockSpec((block_rows, D), lambda i: (i, 0)),
        out_shape=jax.ShapeDtypeStruct((N, D), jnp.float32),
    )(acc, acc, h2, deg2t, b2d)


def kernel(x, edge_index, W, b):
    N, D_in = x.shape
    D_out = W.shape[1]
    E = edge_index.shape[1]

    n_per_tile = -(-N // (NS * CW)) * CW
    N_pad = NS * n_per_tile
    if N_pad == N:
        N_pad += NS * CW
        n_per_tile = N_pad // NS

    CH = -(-E // (NW * CW))
    E_pad = NW * CH * CW
    pad = E_pad - E
    src = edge_index[0]
    dst = edge_index[1]
    if pad:
        fill = jnp.arange(pad, dtype=jnp.int32)
        src = jnp.concatenate([src, fill % N])
        dst = jnp.concatenate([dst, N + fill % (N_pad - N)])
    src_r = src.reshape(NW, CH, CW)
    dst_r = dst.reshape(NW, CH, CW)

    x_pad = jnp.pad(x, ((0, N_pad - N), (0, 0)))

    deg2 = _make_deg_kernel(N_pad, CH)(dst_r)
    deg2t = deg2.T
    h2 = _prep_call(x_pad, W, deg2t, 256)
    acc = _make_agg_kernel(N_pad, CH, D_out)(src_r, dst_r, h2)
    out = _combine_call(acc, h2, deg2t, b.reshape(1, D_out), N, 256)
    return out


# speedup vs baseline: 33.8017x; 1.1814x over previous
import functools

import jax
import jax.numpy as jnp
from jax import lax
from jax.experimental import pallas as pl
from jax.experimental.pallas import tpu as pltpu
from jax.experimental.pallas import tpu_sc as plsc

NC = 2
NS = 16
NL = 16
NW = NC * NS
CW = 128
NPH = 2


def _sc_mesh():
    return plsc.VectorSubcoreMesh(
        core_axis_name="c", subcore_axis_name="s", num_cores=NC, num_subcores=NS
    )


def _make_deg_kernel(N_pad, CH):
    n_per_tile = N_pad // NS

    @functools.partial(
        pl.kernel,
        out_type=jax.ShapeDtypeStruct((NC, N_pad), jnp.float32),
        mesh=_sc_mesh(),
        scratch_types=[
            pltpu.VMEM((CH, CW), jnp.int32),
            pltpu.VMEM((CW,), jnp.float32),
            pltpu.VMEM((n_per_tile,), jnp.float32),
            pltpu.VMEM_SHARED((N_pad,), jnp.float32),
        ],
    )
    def deg_kernel(dst_hbm, out_hbm, idx_v, ones_v, zeros_v, acc_sh):
        c = lax.axis_index("c")
        s = lax.axis_index("s")
        wid = c * NS + s

        def fill_ones(i, _):
            ones_v[pl.ds(i * NL, NL)] = jnp.ones((NL,), jnp.float32)
            return 0

        lax.fori_loop(0, CW // NL, fill_ones, 0)

        def fill_zeros(i, _):
            zeros_v[pl.ds(i * NL, NL)] = jnp.zeros((NL,), jnp.float32)
            return 0

        lax.fori_loop(0, n_per_tile // NL, fill_zeros, 0)
        pltpu.sync_copy(zeros_v, acc_sh.at[pl.ds(s * n_per_tile, n_per_tile)])
        pltpu.sync_copy(dst_hbm.at[wid], idx_v)
        plsc.subcore_barrier()

        def body(j, _):
            pltpu.sync_copy(ones_v, acc_sh.at[idx_v.at[j]], add=True)
            return 0

        lax.fori_loop(0, CH, body, 0)
        plsc.subcore_barrier()
        sl = pl.ds(s * n_per_tile, n_per_tile)
        pltpu.sync_copy(acc_sh.at[sl], out_hbm.at[c, sl])

    return deg_kernel


def _make_agg_kernel(N_pad, CH, D):
    n_per_tile = N_pad // NS
    PH = CH // NPH

    @functools.partial(
        pl.kernel,
        out_type=jax.ShapeDtypeStruct((NC, N_pad, D), jnp.float32),
        mesh=_sc_mesh(),
        scratch_types=[
            pltpu.VMEM((PH, CW), jnp.int32),
            pltpu.VMEM((PH, CW), jnp.int32),
            pltpu.VMEM((CW, D), jnp.float32),
            pltpu.VMEM((CW, D), jnp.float32),
            pltpu.VMEM_SHARED((N_pad, D), jnp.float32),
            pltpu.SemaphoreType.DMA,
        ],
    )
    def agg_kernel(src_hbm, dst_hbm, h2_hbm, out_hbm, si_v, di_v, rows0_v,
                   rows1_v, acc_sh, sem):
        c = lax.axis_index("c")
        s = lax.axis_index("s")
        wid = c * NS + s
        cols = D // NL

        def fill_zeros(i, _):
            rows0_v[i // cols, pl.ds((i % cols) * NL, NL)] = jnp.zeros(
                (NL,), jnp.float32)
            return 0

        lax.fori_loop(0, CW * cols, fill_zeros, 0)

        def zero_acc(i, _):
            pltpu.sync_copy(
                rows0_v, acc_sh.at[pl.ds(s * n_per_tile + i * CW, CW)])
            return 0

        lax.fori_loop(0, n_per_tile // CW, zero_acc, 0)
        plsc.subcore_barrier()

        for ph in range(NPH):
            pltpu.sync_copy(src_hbm.at[wid, pl.ds(ph * PH, PH)], si_v)
            pltpu.sync_copy(dst_hbm.at[wid, pl.ds(ph * PH, PH)], di_v)

            pltpu.async_copy(h2_hbm.at[si_v.at[0]], rows0_v, sem)

            def pair(jj, _):
                j0 = jj * 2
                j1 = j0 + 1
                pltpu.make_async_copy(
                    h2_hbm.at[si_v.at[j0]], rows0_v, sem).wait()
                pltpu.async_copy(h2_hbm.at[si_v.at[j1]], rows1_v, sem)
                pltpu.sync_copy(rows0_v, acc_sh.at[di_v.at[j0]], add=True)
                pltpu.make_async_copy(
                    h2_hbm.at[si_v.at[j1]], rows1_v, sem).wait()

                @pl.when(j1 + 1 < PH)
                def _():
                    pltpu.async_copy(h2_hbm.at[si_v.at[j1 + 1]], rows0_v, sem)

                pltpu.sync_copy(rows1_v, acc_sh.at[di_v.at[j1]], add=True)
                return 0

            lax.fori_loop(0, PH // 2, pair, 0)

        plsc.subcore_barrier()
        sl = pl.ds(s * n_per_tile, n_per_tile)
        pltpu.sync_copy(acc_sh.at[sl], out_hbm.at[c, sl])

    return agg_kernel


def _prep_call(x_pad, W, deg2t, block_rows):
    N_pad, D_in = x_pad.shape
    D_out = W.shape[1]

    def body(x_ref, w_ref, deg_ref, h2_ref):
        deg = deg_ref[:, 0:1] + deg_ref[:, 1:2] + 1.0
        dis = lax.rsqrt(deg)
        h = jnp.dot(x_ref[...], w_ref[...], preferred_element_type=jnp.float32)
        h2_ref[...] = h * dis

    return pl.pallas_call(
        body,
        grid=(N_pad // block_rows,),
        in_specs=[
            pl.BlockSpec((block_rows, D_in), lambda i: (i, 0)),
            pl.BlockSpec((D_in, D_out), lambda i: (0, 0)),
            pl.BlockSpec((block_rows, NC), lambda i: (i, 0)),
        ],
        out_specs=pl.BlockSpec((block_rows, D_out), lambda i: (i, 0)),
        out_shape=jax.ShapeDtypeStruct((N_pad, D_out), jnp.float32),
    )(x_pad, W, deg2t)


def _combine_call(acc, h2, deg2t, b2d, N, block_rows):
    _, N_pad, D = acc.shape

    def body(a0_ref, a1_ref, h2_ref, deg_ref, b_ref, o_ref):
        deg = deg_ref[:, 0:1] + deg_ref[:, 1:2] + 1.0
        dis = lax.rsqrt(deg)
        o_ref[...] = dis * (a0_ref[0] + a1_ref[0] + h2_ref[...]) + b_ref[...]

    return pl.pallas_call(
        body,
        grid=(N_pad // block_rows,),
        in_specs=[
            pl.BlockSpec((1, block_rows, D), lambda i: (0, i, 0)),
            pl.BlockSpec((1, block_rows, D), lambda i: (1, i, 0)),
            pl.BlockSpec((block_rows, D), lambda i: (i, 0)),
            pl.BlockSpec((block_rows, NC), lambda i: (i, 0)),
            pl.BlockSpec((1, D), lambda i: (0, 0)),
        ],
        out_specs=pl.BlockSpec((block_rows, D), lambda i: (i, 0)),
        out_shape=jax.ShapeDtypeStruct((N, D), jnp.float32),
    )(acc, acc, h2, deg2t, b2d)


def kernel(x, edge_index, W, b):
    N, D_in = x.shape
    D_out = W.shape[1]
    E = edge_index.shape[1]

    n_per_tile = -(-N // (NS * CW)) * CW
    N_pad = NS * n_per_tile
    if N_pad == N:
        N_pad += NS * CW
        n_per_tile = N_pad // NS

    CH = 2 * NPH * -(-E // (NW * CW * 2 * NPH))
    E_pad = NW * CH * CW
    pad = E_pad - E
    src = edge_index[0]
    dst = edge_index[1]
    if pad:
        fill = jnp.arange(pad, dtype=jnp.int32)
        src = jnp.concatenate([src, fill % N])
        dst = jnp.concatenate([dst, N + fill % (N_pad - N)])
    src_r = src.reshape(NW, CH, CW)
    dst_r = dst.reshape(NW, CH, CW)

    x_pad = jnp.pad(x, ((0, N_pad - N), (0, 0)))

    deg2 = _make_deg_kernel(N_pad, CH)(dst_r)
    deg2t = deg2.T
    h2 = _prep_call(x_pad, W, deg2t, 256)
    acc = _make_agg_kernel(N_pad, CH, D_out)(src_r, dst_r, h2)
    out = _combine_call(acc, h2, deg2t, b.reshape(1, D_out), N, 256)
    return out


# speedup vs baseline: 38.7594x; 1.1467x over previous
import functools

import jax
import jax.numpy as jnp
from jax import lax
from jax.experimental import pallas as pl
from jax.experimental.pallas import tpu as pltpu
from jax.experimental.pallas import tpu_sc as plsc

NC = 2
NS = 16
NL = 16
NW = NC * NS
CW = 128
NPH = 2


def _sc_mesh():
    return plsc.VectorSubcoreMesh(
        core_axis_name="c", subcore_axis_name="s", num_cores=NC, num_subcores=NS
    )


def _make_deg_kernel(N_pad, CH):
    n_per_tile = N_pad // NS

    @functools.partial(
        pl.kernel,
        out_type=jax.ShapeDtypeStruct((NC, N_pad), jnp.float32),
        mesh=_sc_mesh(),
        scratch_types=[
            pltpu.VMEM((CH, CW), jnp.int32),
            pltpu.VMEM((CW,), jnp.float32),
            pltpu.VMEM((n_per_tile,), jnp.float32),
            pltpu.VMEM_SHARED((N_pad,), jnp.float32),
        ],
    )
    def deg_kernel(dst_hbm, out_hbm, idx_v, ones_v, zeros_v, acc_sh):
        c = lax.axis_index("c")
        s = lax.axis_index("s")
        wid = c * NS + s

        def fill_ones(i, _):
            ones_v[pl.ds(i * NL, NL)] = jnp.ones((NL,), jnp.float32)
            return 0

        lax.fori_loop(0, CW // NL, fill_ones, 0)

        def fill_zeros(i, _):
            zeros_v[pl.ds(i * NL, NL)] = jnp.zeros((NL,), jnp.float32)
            return 0

        lax.fori_loop(0, n_per_tile // NL, fill_zeros, 0)
        pltpu.sync_copy(zeros_v, acc_sh.at[pl.ds(s * n_per_tile, n_per_tile)])
        pltpu.sync_copy(dst_hbm.at[wid], idx_v)
        plsc.subcore_barrier()

        def body(j, _):
            pltpu.sync_copy(ones_v, acc_sh.at[idx_v.at[j]], add=True)
            return 0

        lax.fori_loop(0, CH, body, 0)
        plsc.subcore_barrier()
        sl = pl.ds(s * n_per_tile, n_per_tile)
        pltpu.sync_copy(acc_sh.at[sl], out_hbm.at[c, sl])

    return deg_kernel


def _make_agg_kernel(N_pad, CH, D):
    n_per_tile = N_pad // NS
    PH = CH // NPH

    @functools.partial(
        pl.kernel,
        out_type=jax.ShapeDtypeStruct((NC, N_pad, D), jnp.float32),
        mesh=_sc_mesh(),
        scratch_types=[
            pltpu.VMEM((PH, CW), jnp.int32),
            pltpu.VMEM((PH, CW), jnp.int32),
            pltpu.VMEM((CW, D), jnp.float32),
            pltpu.VMEM((CW, D), jnp.float32),
            pltpu.VMEM_SHARED((N_pad, D), jnp.float32),
            pltpu.SemaphoreType.DMA,
            pltpu.SemaphoreType.DMA,
        ],
    )
    def agg_kernel(src_hbm, dst_hbm, h2_hbm, out_hbm, si_v, di_v, rows0_v,
                   rows1_v, acc_sh, gsem, ssem):
        c = lax.axis_index("c")
        s = lax.axis_index("s")
        wid = c * NS + s
        cols = D // NL

        def fill_zeros(i, _):
            rows0_v[i // cols, pl.ds((i % cols) * NL, NL)] = jnp.zeros(
                (NL,), jnp.float32)
            return 0

        lax.fori_loop(0, CW * cols, fill_zeros, 0)

        def zero_acc(i, _):
            pltpu.sync_copy(
                rows0_v, acc_sh.at[pl.ds(s * n_per_tile + i * CW, CW)])
            return 0

        lax.fori_loop(0, n_per_tile // CW, zero_acc, 0)
        plsc.subcore_barrier()

        for ph in range(NPH):
            pltpu.sync_copy(src_hbm.at[wid, pl.ds(ph * PH, PH)], si_v)
            pltpu.sync_copy(dst_hbm.at[wid, pl.ds(ph * PH, PH)], di_v)

            pltpu.async_copy(h2_hbm.at[si_v.at[0]], rows0_v, gsem)

            def pair(jj, _):
                j0 = jj * 2
                j1 = j0 + 1
                pltpu.make_async_copy(
                    h2_hbm.at[si_v.at[j0]], rows0_v, gsem).wait()
                pltpu.async_copy(rows0_v, acc_sh.at[di_v.at[j0]], ssem,
                                 add=True)

                @pl.when(jj >= 1)
                def _():
                    pltpu.make_async_copy(
                        rows1_v, acc_sh.at[di_v.at[j0 - 1]], ssem).wait()

                pltpu.async_copy(h2_hbm.at[si_v.at[j1]], rows1_v, gsem)
                pltpu.make_async_copy(
                    h2_hbm.at[si_v.at[j1]], rows1_v, gsem).wait()
                pltpu.async_copy(rows1_v, acc_sh.at[di_v.at[j1]], ssem,
                                 add=True)
                pltpu.make_async_copy(
                    rows0_v, acc_sh.at[di_v.at[j0]], ssem).wait()

                @pl.when(j1 + 1 < PH)
                def _():
                    pltpu.async_copy(h2_hbm.at[si_v.at[j1 + 1]], rows0_v, gsem)

                return 0

            lax.fori_loop(0, PH // 2, pair, 0)
            pltpu.make_async_copy(
                rows1_v, acc_sh.at[di_v.at[PH - 1]], ssem).wait()

        plsc.subcore_barrier()
        sl = pl.ds(s * n_per_tile, n_per_tile)
        pltpu.sync_copy(acc_sh.at[sl], out_hbm.at[c, sl])

    return agg_kernel


def _prep_call(x_pad, W, deg2t, block_rows):
    N_pad, D_in = x_pad.shape
    D_out = W.shape[1]

    def body(x_ref, w_ref, deg_ref, h2_ref):
        deg = deg_ref[:, 0:1] + deg_ref[:, 1:2] + 1.0
        dis = lax.rsqrt(deg)
        h = jnp.dot(x_ref[...], w_ref[...], preferred_element_type=jnp.float32)
        h2_ref[...] = h * dis

    return pl.pallas_call(
        body,
        grid=(N_pad // block_rows,),
        in_specs=[
            pl.BlockSpec((block_rows, D_in), lambda i: (i, 0)),
            pl.BlockSpec((D_in, D_out), lambda i: (0, 0)),
            pl.BlockSpec((block_rows, NC), lambda i: (i, 0)),
        ],
        out_specs=pl.BlockSpec((block_rows, D_out), lambda i: (i, 0)),
        out_shape=jax.ShapeDtypeStruct((N_pad, D_out), jnp.float32),
    )(x_pad, W, deg2t)


def _combine_call(acc, h2, deg2t, b2d, N, block_rows):
    _, N_pad, D = acc.shape

    def body(a0_ref, a1_ref, h2_ref, deg_ref, b_ref, o_ref):
        deg = deg_ref[:, 0:1] + deg_ref[:, 1:2] + 1.0
        dis = lax.rsqrt(deg)
        o_ref[...] = dis * (a0_ref[0] + a1_ref[0] + h2_ref[...]) + b_ref[...]

    return pl.pallas_call(
        body,
        grid=(N_pad // block_rows,),
        in_specs=[
            pl.BlockSpec((1, block_rows, D), lambda i: (0, i, 0)),
            pl.BlockSpec((1, block_rows, D), lambda i: (1, i, 0)),
            pl.BlockSpec((block_rows, D), lambda i: (i, 0)),
            pl.BlockSpec((block_rows, NC), lambda i: (i, 0)),
            pl.BlockSpec((1, D), lambda i: (0, 0)),
        ],
        out_specs=pl.BlockSpec((block_rows, D), lambda i: (i, 0)),
        out_shape=jax.ShapeDtypeStruct((N, D), jnp.float32),
    )(acc, acc, h2, deg2t, b2d)


def kernel(x, edge_index, W, b):
    N, D_in = x.shape
    D_out = W.shape[1]
    E = edge_index.shape[1]

    n_per_tile = -(-N // (NS * CW)) * CW
    N_pad = NS * n_per_tile
    if N_pad == N:
        N_pad += NS * CW
        n_per_tile = N_pad // NS

    CH = 2 * NPH * -(-E // (NW * CW * 2 * NPH))
    E_pad = NW * CH * CW
    pad = E_pad - E
    src = edge_index[0]
    dst = edge_index[1]
    if pad:
        trash = N_pad - N
        reps = -(-pad // trash)
        fill_dst = (N + jnp.broadcast_to(
            jnp.arange(trash, dtype=jnp.int32), (reps, trash)).reshape(-1))
        fill_src = jnp.minimum(jnp.arange(pad, dtype=jnp.int32), N - 1)
        src = jnp.concatenate([src, fill_src])
        dst = jnp.concatenate([dst, fill_dst[:pad]])
    src_r = src.reshape(NW, CH, CW)
    dst_r = dst.reshape(NW, CH, CW)

    x_pad = jnp.pad(x, ((0, N_pad - N), (0, 0)))

    deg2 = _make_deg_kernel(N_pad, CH)(dst_r)
    deg2t = deg2.T
    h2 = _prep_call(x_pad, W, deg2t, 1024)
    acc = _make_agg_kernel(N_pad, CH, D_out)(src_r, dst_r, h2)
    out = _combine_call(acc, h2, deg2t, b.reshape(1, D_out), N, 1024)
    return out


# speedup vs baseline: 41.0198x; 1.0583x over previous
import functools

import jax
import jax.numpy as jnp
import numpy as np
from jax import lax
from jax.experimental import pallas as pl
from jax.experimental.pallas import tpu as pltpu
from jax.experimental.pallas import tpu_sc as plsc

NC = 2
NS = 16
NL = 16
NW = NC * NS
CW = 128


def _sc_mesh():
    return plsc.VectorSubcoreMesh(
        core_axis_name="c", subcore_axis_name="s", num_cores=NC, num_subcores=NS
    )


def _make_deg_kernel(N_pad, CB):
    n_per_tile = N_pad // NS

    @functools.partial(
        pl.kernel,
        out_type=jax.ShapeDtypeStruct((NC, N_pad), jnp.float32),
        mesh=_sc_mesh(),
        scratch_types=[
            pltpu.VMEM((CB, CW), jnp.int32),
            pltpu.VMEM((CW,), jnp.float32),
            pltpu.VMEM((n_per_tile,), jnp.float32),
            pltpu.VMEM_SHARED((N_pad,), jnp.float32),
        ],
    )
    def deg_kernel(edges_hbm, out_hbm, idx_v, ones_v, zeros_v, acc_sh):
        c = lax.axis_index("c")
        s = lax.axis_index("s")
        wid = c * NS + s

        def fill_ones(i, _):
            ones_v[pl.ds(i * NL, NL)] = jnp.ones((NL,), jnp.float32)
            return 0

        lax.fori_loop(0, CW // NL, fill_ones, 0)

        def fill_zeros(i, _):
            zeros_v[pl.ds(i * NL, NL)] = jnp.zeros((NL,), jnp.float32)
            return 0

        lax.fori_loop(0, n_per_tile // NL, fill_zeros, 0)
        pltpu.sync_copy(zeros_v, acc_sh.at[pl.ds(s * n_per_tile, n_per_tile)])
        pltpu.sync_copy(edges_hbm.at[1, pl.ds(wid * CB, CB)], idx_v)
        plsc.subcore_barrier()

        def body(j, _):
            pltpu.sync_copy(ones_v, acc_sh.at[idx_v.at[j]], add=True)
            return 0

        lax.fori_loop(0, CB, body, 0)
        plsc.subcore_barrier()
        sl = pl.ds(s * n_per_tile, n_per_tile)
        pltpu.sync_copy(acc_sh.at[sl], out_hbm.at[c, sl])

    return deg_kernel


def _make_agg_kernel(N_pad, CB, D, PH):
    n_per_tile = N_pad // NS
    assert CB % PH == 0 and PH % 2 == 0

    @functools.partial(
        pl.kernel,
        out_type=jax.ShapeDtypeStruct((NC, N_pad, D), jnp.float32),
        mesh=_sc_mesh(),
        scratch_types=[
            pltpu.VMEM((PH, CW), jnp.int32),
            pltpu.VMEM((PH, CW), jnp.int32),
            pltpu.VMEM((CW, D), jnp.float32),
            pltpu.VMEM((CW, D), jnp.float32),
            pltpu.VMEM_SHARED((N_pad, D), jnp.float32),
            pltpu.SemaphoreType.DMA,
            pltpu.SemaphoreType.DMA,
        ],
    )
    def agg_kernel(edges_hbm, h2_hbm, out_hbm, si_v, di_v, rows0_v,
                   rows1_v, acc_sh, gsem, ssem):
        c = lax.axis_index("c")
        s = lax.axis_index("s")
        wid = c * NS + s
        cols = D // NL

        def fill_zeros(i, _):
            rows0_v[i // cols, pl.ds((i % cols) * NL, NL)] = jnp.zeros(
                (NL,), jnp.float32)
            return 0

        lax.fori_loop(0, CW * cols, fill_zeros, 0)

        def zero_acc(i, _):
            pltpu.sync_copy(
                rows0_v, acc_sh.at[pl.ds(s * n_per_tile + i * CW, CW)])
            return 0

        lax.fori_loop(0, n_per_tile // CW, zero_acc, 0)
        plsc.subcore_barrier()

        for ph in range(CB // PH):
            base = wid * CB + ph * PH
            pltpu.sync_copy(edges_hbm.at[0, pl.ds(base, PH)], si_v)
            pltpu.sync_copy(edges_hbm.at[1, pl.ds(base, PH)], di_v)

            pltpu.async_copy(h2_hbm.at[si_v.at[0]], rows0_v, gsem)

            def pair(jj, _):
                j0 = jj * 2
                j1 = j0 + 1
                pltpu.make_async_copy(
                    h2_hbm.at[si_v.at[j0]], rows0_v, gsem).wait()
                pltpu.async_copy(rows0_v, acc_sh.at[di_v.at[j0]], ssem,
                                 add=True)

                @pl.when(jj >= 1)
                def _():
                    pltpu.make_async_copy(
                        rows1_v, acc_sh.at[di_v.at[j0 - 1]], ssem).wait()

                pltpu.async_copy(h2_hbm.at[si_v.at[j1]], rows1_v, gsem)
                pltpu.make_async_copy(
                    h2_hbm.at[si_v.at[j1]], rows1_v, gsem).wait()
                pltpu.async_copy(rows1_v, acc_sh.at[di_v.at[j1]], ssem,
                                 add=True)
                pltpu.make_async_copy(
                    rows0_v, acc_sh.at[di_v.at[j0]], ssem).wait()

                @pl.when(j1 + 1 < PH)
                def _():
                    pltpu.async_copy(h2_hbm.at[si_v.at[j1 + 1]], rows0_v, gsem)

                return 0

            lax.fori_loop(0, PH // 2, pair, 0)
            pltpu.make_async_copy(
                rows1_v, acc_sh.at[di_v.at[PH - 1]], ssem).wait()

        plsc.subcore_barrier()
        sl = pl.ds(s * n_per_tile, n_per_tile)
        pltpu.sync_copy(acc_sh.at[sl], out_hbm.at[c, sl])

    return agg_kernel


def _prep_call(x, W, deg2t, N_pad, block_rows):
    N, D_in = x.shape
    D_out = W.shape[1]

    def body(x_ref, w_ref, deg_ref, h2_ref):
        deg = deg_ref[:, 0:1] + deg_ref[:, 1:2] + 1.0
        dis = lax.rsqrt(deg)
        h = jnp.dot(x_ref[...], w_ref[...], preferred_element_type=jnp.float32)
        h2_ref[...] = h * dis

    return pl.pallas_call(
        body,
        grid=(N_pad // block_rows,),
        in_specs=[
            pl.BlockSpec((block_rows, D_in), lambda i: (i, 0)),
            pl.BlockSpec((D_in, D_out), lambda i: (0, 0)),
            pl.BlockSpec((block_rows, NC), lambda i: (i, 0)),
        ],
        out_specs=pl.BlockSpec((block_rows, D_out), lambda i: (i, 0)),
        out_shape=jax.ShapeDtypeStruct((N_pad, D_out), jnp.float32),
    )(x, W, deg2t)


def _combine_call(acc, h2, deg2t, b2d, N, block_rows):
    _, N_pad, D = acc.shape

    def body(a0_ref, a1_ref, h2_ref, deg_ref, b_ref, o_ref):
        deg = deg_ref[:, 0:1] + deg_ref[:, 1:2] + 1.0
        dis = lax.rsqrt(deg)
        o_ref[...] = dis * (a0_ref[0] + a1_ref[0] + h2_ref[...]) + b_ref[...]

    return pl.pallas_call(
        body,
        grid=(N_pad // block_rows,),
        in_specs=[
            pl.BlockSpec((1, block_rows, D), lambda i: (0, i, 0)),
            pl.BlockSpec((1, block_rows, D), lambda i: (1, i, 0)),
            pl.BlockSpec((block_rows, D), lambda i: (i, 0)),
            pl.BlockSpec((block_rows, NC), lambda i: (i, 0)),
            pl.BlockSpec((1, D), lambda i: (0, 0)),
        ],
        out_specs=pl.BlockSpec((block_rows, D), lambda i: (i, 0)),
        out_shape=jax.ShapeDtypeStruct((N, D), jnp.float32),
    )(acc, acc, h2, deg2t, b2d)


def kernel(x, edge_index, W, b):
    N, D_in = x.shape
    D_out = W.shape[1]
    E = edge_index.shape[1]

    n_per_tile = -(-N // (NS * CW)) * CW
    N_pad = NS * n_per_tile
    if N_pad == N:
        N_pad += NS * CW
        n_per_tile = N_pad // NS

    assert E % CW == 0, "edge count must be a multiple of the chunk width"
    R = E // CW
    CB = 8 * -(-R // (NW * 8))
    R_pad = NW * CB
    edges_r = edge_index.reshape(2, R, CW)
    if R_pad > R:
        nfill = (R_pad - R) * CW
        fill_src = np.arange(nfill, dtype=np.int64) % N
        fill_dst = N + np.arange(nfill, dtype=np.int64) % (N_pad - N)
        fills = np.stack([fill_src.reshape(R_pad - R, CW),
                          fill_dst.reshape(R_pad - R, CW)]).astype(np.int32)
        edges_r = jnp.concatenate([edges_r, jnp.asarray(fills)], axis=1)

    deg2 = _make_deg_kernel(N_pad, CB)(edges_r)
    deg2t = deg2.T
    h2 = _prep_call(x, W, deg2t, N_pad, 1024)
    acc = _make_agg_kernel(N_pad, CB, D_out, CB // 2)(edges_r, h2)
    out = _combine_call(acc, h2, deg2t, b.reshape(1, D_out), N, 1024)
    return out


# speedup vs baseline: 43.1119x; 1.0510x over previous
import functools

import jax
import jax.numpy as jnp
import numpy as np
from jax import lax
from jax.experimental import pallas as pl
from jax.experimental.pallas import tpu as pltpu
from jax.experimental.pallas import tpu_sc as plsc

NC = 2
NS = 16
NL = 16
NW = NC * NS
CW = 128


def _sc_mesh():
    return plsc.VectorSubcoreMesh(
        core_axis_name="c", subcore_axis_name="s", num_cores=NC, num_subcores=NS
    )


def _make_deg_kernel(N_pad, CB):
    n_per_tile = N_pad // NS

    @functools.partial(
        pl.kernel,
        out_type=jax.ShapeDtypeStruct((NC, N_pad), jnp.float32),
        mesh=_sc_mesh(),
        scratch_types=[
            pltpu.VMEM((CB, CW), jnp.int32),
            pltpu.VMEM((CW,), jnp.float32),
            pltpu.VMEM((n_per_tile,), jnp.float32),
            pltpu.VMEM_SHARED((N_pad,), jnp.float32),
            pltpu.SemaphoreType.DMA,
        ],
    )
    def deg_kernel(edges_hbm, out_hbm, idx_v, ones_v, zeros_v, acc_sh, sem):
        c = lax.axis_index("c")
        s = lax.axis_index("s")
        wid = c * NS + s

        def fill_ones(i, _):
            ones_v[pl.ds(i * NL, NL)] = jnp.ones((NL,), jnp.float32)
            return 0

        lax.fori_loop(0, CW // NL, fill_ones, 0)

        def fill_zeros(i, _):
            zeros_v[pl.ds(i * NL, NL)] = jnp.zeros((NL,), jnp.float32)
            return 0

        lax.fori_loop(0, n_per_tile // NL, fill_zeros, 0)
        pltpu.sync_copy(zeros_v, acc_sh.at[pl.ds(s * n_per_tile, n_per_tile)])
        pltpu.sync_copy(edges_hbm.at[1, pl.ds(wid * CB, CB)], idx_v)
        plsc.subcore_barrier()

        W_IN_FLIGHT = 8

        def body(j, _):
            pltpu.async_copy(ones_v, acc_sh.at[idx_v.at[j]], sem, add=True)

            @pl.when(j >= W_IN_FLIGHT)
            def _():
                pltpu.make_async_copy(
                    ones_v, acc_sh.at[idx_v.at[j - W_IN_FLIGHT]], sem).wait()

            return 0

        lax.fori_loop(0, CB, body, 0)

        drain_start = max(0, CB - W_IN_FLIGHT)

        def drain(j, _):
            pltpu.make_async_copy(
                ones_v, acc_sh.at[idx_v.at[drain_start + j]], sem).wait()
            return 0

        lax.fori_loop(0, CB - drain_start, drain, 0)
        plsc.subcore_barrier()
        sl = pl.ds(s * n_per_tile, n_per_tile)
        pltpu.sync_copy(acc_sh.at[sl], out_hbm.at[c, sl])

    return deg_kernel


def _make_agg_kernel(N_pad, CB, D, PH):
    n_per_tile = N_pad // NS
    assert CB % PH == 0 and PH % 2 == 0

    @functools.partial(
        pl.kernel,
        out_type=jax.ShapeDtypeStruct((NC, N_pad, D), jnp.float32),
        mesh=_sc_mesh(),
        scratch_types=[
            pltpu.VMEM((PH, CW), jnp.int32),
            pltpu.VMEM((PH, CW), jnp.int32),
            pltpu.VMEM((CW, D), jnp.float32),
            pltpu.VMEM((CW, D), jnp.float32),
            pltpu.VMEM_SHARED((N_pad, D), jnp.float32),
            pltpu.SemaphoreType.DMA,
            pltpu.SemaphoreType.DMA,
        ],
    )
    def agg_kernel(edges_hbm, h2_hbm, out_hbm, si_v, di_v, rows0_v,
                   rows1_v, acc_sh, gsem, ssem):
        c = lax.axis_index("c")
        s = lax.axis_index("s")
        wid = c * NS + s
        cols = D // NL

        def fill_zeros(i, _):
            rows0_v[i // cols, pl.ds((i % cols) * NL, NL)] = jnp.zeros(
                (NL,), jnp.float32)
            return 0

        lax.fori_loop(0, CW * cols, fill_zeros, 0)

        def zero_acc(i, _):
            pltpu.sync_copy(
                rows0_v, acc_sh.at[pl.ds(s * n_per_tile + i * CW, CW)])
            return 0

        lax.fori_loop(0, n_per_tile // CW, zero_acc, 0)
        plsc.subcore_barrier()

        for ph in range(CB // PH):
            base = wid * CB + ph * PH
            pltpu.sync_copy(edges_hbm.at[0, pl.ds(base, PH)], si_v)
            pltpu.sync_copy(edges_hbm.at[1, pl.ds(base, PH)], di_v)

            pltpu.async_copy(h2_hbm.at[si_v.at[0]], rows0_v, gsem)

            def pair(jj, _):
                j0 = jj * 2
                j1 = j0 + 1
                pltpu.make_async_copy(
                    h2_hbm.at[si_v.at[j0]], rows0_v, gsem).wait()
                pltpu.async_copy(rows0_v, acc_sh.at[di_v.at[j0]], ssem,
                                 add=True)

                @pl.when(jj >= 1)
                def _():
                    pltpu.make_async_copy(
                        rows1_v, acc_sh.at[di_v.at[j0 - 1]], ssem).wait()

                pltpu.async_copy(h2_hbm.at[si_v.at[j1]], rows1_v, gsem)
                pltpu.make_async_copy(
                    h2_hbm.at[si_v.at[j1]], rows1_v, gsem).wait()
                pltpu.async_copy(rows1_v, acc_sh.at[di_v.at[j1]], ssem,
                                 add=True)
                pltpu.make_async_copy(
                    rows0_v, acc_sh.at[di_v.at[j0]], ssem).wait()

                @pl.when(j1 + 1 < PH)
                def _():
                    pltpu.async_copy(h2_hbm.at[si_v.at[j1 + 1]], rows0_v, gsem)

                return 0

            lax.fori_loop(0, PH // 2, pair, 0)
            pltpu.make_async_copy(
                rows1_v, acc_sh.at[di_v.at[PH - 1]], ssem).wait()

        plsc.subcore_barrier()
        sl = pl.ds(s * n_per_tile, n_per_tile)
        pltpu.sync_copy(acc_sh.at[sl], out_hbm.at[c, sl])

    return agg_kernel


def _prep_call(x, W, deg2t, N_pad, block_rows):
    N, D_in = x.shape
    D_out = W.shape[1]

    def body(x_ref, w_ref, deg_ref, h2_ref):
        deg = deg_ref[:, 0:1] + deg_ref[:, 1:2] + 1.0
        dis = lax.rsqrt(deg)
        h = jnp.dot(x_ref[...], w_ref[...], preferred_element_type=jnp.float32)
        h2_ref[...] = h * dis

    return pl.pallas_call(
        body,
        grid=(N_pad // block_rows,),
        in_specs=[
            pl.BlockSpec((block_rows, D_in), lambda i: (i, 0)),
            pl.BlockSpec((D_in, D_out), lambda i: (0, 0)),
            pl.BlockSpec((block_rows, NC), lambda i: (i, 0)),
        ],
        out_specs=pl.BlockSpec((block_rows, D_out), lambda i: (i, 0)),
        out_shape=jax.ShapeDtypeStruct((N_pad, D_out), jnp.float32),
    )(x, W, deg2t)


def _combine_call(acc, h2, deg2t, b2d, N, block_rows):
    _, N_pad, D = acc.shape

    def body(a0_ref, a1_ref, h2_ref, deg_ref, b_ref, o_ref):
        deg = deg_ref[:, 0:1] + deg_ref[:, 1:2] + 1.0
        dis = lax.rsqrt(deg)
        o_ref[...] = dis * (a0_ref[0] + a1_ref[0] + h2_ref[...]) + b_ref[...]

    return pl.pallas_call(
        body,
        grid=(N_pad // block_rows,),
        in_specs=[
            pl.BlockSpec((1, block_rows, D), lambda i: (0, i, 0)),
            pl.BlockSpec((1, block_rows, D), lambda i: (1, i, 0)),
            pl.BlockSpec((block_rows, D), lambda i: (i, 0)),
            pl.BlockSpec((block_rows, NC), lambda i: (i, 0)),
            pl.BlockSpec((1, D), lambda i: (0, 0)),
        ],
        out_specs=pl.BlockSpec((block_rows, D), lambda i: (i, 0)),
        out_shape=jax.ShapeDtypeStruct((N, D), jnp.float32),
    )(acc, acc, h2, deg2t, b2d)


def kernel(x, edge_index, W, b):
    N, D_in = x.shape
    D_out = W.shape[1]
    E = edge_index.shape[1]

    n_per_tile = -(-N // (NS * CW)) * CW
    N_pad = NS * n_per_tile
    if N_pad == N:
        N_pad += NS * CW
        n_per_tile = N_pad // NS

    assert E % CW == 0, "edge count must be a multiple of the chunk width"
    R = E // CW
    CB = 8 * -(-R // (NW * 8))
    R_pad = NW * CB
    edges_r = edge_index.reshape(2, R, CW)
    if R_pad > R:
        nfill = (R_pad - R) * CW
        fill_src = np.arange(nfill, dtype=np.int64) % N
        fill_dst = N + np.arange(nfill, dtype=np.int64) % (N_pad - N)
        fills = np.stack([fill_src.reshape(R_pad - R, CW),
                          fill_dst.reshape(R_pad - R, CW)]).astype(np.int32)
        edges_r = jnp.concatenate([edges_r, jnp.asarray(fills)], axis=1)

    deg2 = _make_deg_kernel(N_pad, CB)(edges_r)
    deg2t = deg2.T
    h2 = _prep_call(x, W, deg2t, N_pad, 2048)
    acc = _make_agg_kernel(N_pad, CB, D_out, CB // 2)(edges_r, h2)
    out = _combine_call(acc, h2, deg2t, b.reshape(1, D_out), N, 2048)
    return out
